# baseline (speedup 1.0000x reference)
import functools

import numpy as np
import jax
import jax.numpy as jnp
from jax import lax
from jax.experimental import pallas as pl
from jax.experimental.pallas import tpu as pltpu
from jax.experimental.pallas import tpu_sc as plsc

_CLS_W = 1.0
_REG_W = 1.0
_STRIDE = 8
_SEARCH = 255
_FEAT = 17
_RATIOS = [0.33, 0.5, 1.0, 2.0, 3.0]
_SCALES = [8]
_B = 32
_A = len(_RATIOS) * len(_SCALES)
_L = _A * _FEAT * _FEAT
_LP = 1536
_NPOS = 16
_NNEG_T = 3.0
_MTN = 624
_MTP = 632


def _build_anchor_rows():
    n = _A
    na = np.zeros((n, 4), np.float32)
    size = _STRIDE * _STRIDE
    c = 0
    for r in _RATIOS:
        ws = int(np.sqrt(size / r))
        hs = int(ws * r)
        for s in _SCALES:
            na[c] = [0.0, 0.0, ws * s, hs * s]
            c += 1
    ori = _SEARCH // 2 - (_FEAT // 2) * _STRIDE
    xs = (ori + _STRIDE * np.arange(_FEAT)).astype(np.float32)
    xx, yy = np.meshgrid(xs, xs)
    cx = np.tile(xx[None], (n, 1, 1))
    cy = np.tile(yy[None], (n, 1, 1))
    w = np.tile(na[:, 2, None, None], (1, _FEAT, _FEAT))
    h = np.tile(na[:, 3, None, None], (1, _FEAT, _FEAT))
    corner = np.stack([cx - w / 2, cy - h / 2, cx + w / 2, cy + h / 2], 0).reshape(4, -1)
    center = np.stack([cx, cy, w, h], 0).reshape(4, -1)
    anc = np.zeros((8, _LP), np.float32)
    anc[0:4, :_L] = corner
    anc[4:8, :_L] = center
    anc[6, _L:] = 1.0
    anc[7, _L:] = 1.0
    return anc


def _build_mt_rows():
    rows = np.zeros((_B, _MTP), np.uint32)
    for i in range(_B):
        st = np.random.RandomState(i).get_state()
        rows[i, :_MTN] = st[1].astype(np.uint32)
        rows[i, _MTN] = np.uint32(st[2])
    return rows.view(np.int32)


_ANC = _build_anchor_rows()
_MT_ROWS = _build_mt_rows()


def _sl1(x):
    d = jnp.abs(x)
    return jnp.where(d < 1.0, 0.5 * d * d, d - 0.5)


def _iou_labels(g_ref, anc_ref):
    gx1 = g_ref[:, 0:1]
    gy1 = g_ref[:, 1:2]
    gx2 = g_ref[:, 2:3]
    gy2 = g_ref[:, 3:4]
    ax1 = anc_ref[0:1, :]
    ay1 = anc_ref[1:2, :]
    ax2 = anc_ref[2:3, :]
    ay2 = anc_ref[3:4, :]
    ix = jnp.clip(jnp.minimum(gx2, ax2) - jnp.maximum(gx1, ax1), 0.0, None)
    iy = jnp.clip(jnp.minimum(gy2, ay2) - jnp.maximum(gy1, ay1), 0.0, None)
    inter = ix * iy
    area_g = (gx2 - gx1) * (gy2 - gy1)
    area_a = (ax2 - ax1) * (ay2 - ay1)
    iou = inter / (area_g + area_a - inter)
    return jnp.where(iou > 0.6, 1, jnp.where(iou < 0.3, 0, -1)).astype(jnp.int32)


def _tc_body(cls_ref, reg_ref, g_ref, g4_ref, anc_ref,
             m_ref, lp_ref, ln_ref, lre_ref):
    lab = _iou_labels(g_ref, anc_ref)
    lidx = lax.broadcasted_iota(jnp.int32, (_B, _LP), 1)
    m_ref[...] = jnp.where(lidx < _L, lab, -1)
    c = jnp.clip(cls_ref[...], 1e-7, 1.0 - 1e-7)
    lp_ref[...] = -jnp.log(c)
    ln_ref[...] = -jnp.log(1.0 - c)
    acx = anc_ref[4:5, :]
    acy = anc_ref[5:6, :]
    acw = anc_ref[6:7, :]
    ach = anc_ref[7:8, :]
    k = lax.broadcasted_iota(jnp.int32, (4 * _B, 1), 0) % 4
    keven = k % 2 == 0
    esel = jnp.where(keven, acx, acy)
    wsel = jnp.where(keven, acw, ach)
    gval = g4_ref[...]
    gcol = jnp.where(k == 0, gval[:, 0:1],
                     jnp.where(k == 1, gval[:, 1:2],
                               jnp.where(k == 2, gval[:, 2:3], gval[:, 3:4])))
    enc = (gcol - esel) / wsel
    sl = _sl1(reg_ref[...] - enc)
    lab4 = _iou_labels(g4_ref, anc_ref)
    msl = jnp.where(lab4 == 1, sl, 0.0)
    lre_ref[...] = jnp.broadcast_to(jnp.sum(msl, axis=1, keepdims=True),
                                    (4 * _B, 16))


def _sread(ref, i):
    idxv = jnp.broadcast_to(jnp.asarray(i, jnp.int32), (16,))
    return plsc.load_gather(ref, [idxv])[0]


def _swrite(ref, i, val):
    idxv = jnp.broadcast_to(jnp.asarray(i, jnp.int32), (16,))
    valv = jnp.broadcast_to(val, (16,))
    mask = lax.iota(jnp.int32, 16) == 0
    plsc.store_scatter(ref, [idxv], valv, mask=mask)


def _sc_body(nc, m_hbm, lp_hbm, ln_hbm, lre_hbm, mt_hbm, out_hbm,
             m_v, lp_v, ln_v, pos_v, comb_v, mt_v, st_v, lre4_v, res_v):
    wid = lax.axis_index("s") * nc + lax.axis_index("c")
    pltpu.sync_copy(m_hbm.at[wid], m_v)
    pltpu.sync_copy(lp_hbm.at[wid], lp_v)
    pltpu.sync_copy(ln_hbm.at[wid], ln_v)
    pltpu.sync_copy(lre_hbm.at[pl.ds(4 * wid, 4)], lre4_v)
    pltpu.sync_copy(mt_hbm.at[wid], mt_v)
    lre_num = (lre4_v[0] + lre4_v[1] + lre4_v[2] + lre4_v[3])[0]

    lane = lax.iota(jnp.int32, 16)
    u1 = jnp.uint32(1)
    up = jnp.uint32(0x80000000)
    lo = jnp.uint32(0x7FFFFFFF)
    mag = jnp.uint32(0x9908B0DF)
    z = jnp.uint32(0)
    nch = _MTN // 16

    def temper_all():
        def tp(c, carry):
            y = mt_v[pl.ds(c * 16, 16)].astype(jnp.uint32)
            y = y ^ (y >> jnp.uint32(11))
            y = y ^ ((y << jnp.uint32(7)) & jnp.uint32(0x9D2C5680))
            y = y ^ ((y << jnp.uint32(15)) & jnp.uint32(0xEFC60000))
            y = y ^ (y >> jnp.uint32(18))
            st_v[pl.ds(c * 16, 16)] = y.astype(jnp.int32)
            return carry
        lax.fori_loop(0, nch, tp, 0)

    def twist():
        def tb(c, carry):
            base = c * 16
            iv = base + lane
            cur = mt_v[pl.ds(base, 16)].astype(jnp.uint32)
            nxt = jnp.where(iv == _MTN - 1, 0, iv + 1)
            far = jnp.where(iv < _MTN - 397, iv + 397, iv - (_MTN - 397))
            nv = plsc.load_gather(mt_v, [nxt]).astype(jnp.uint32)
            fv = plsc.load_gather(mt_v, [far]).astype(jnp.uint32)
            y = (cur & up) | (nv & lo)
            v = fv ^ (y >> u1) ^ jnp.where((y & u1) != z, mag, z)
            mt_v[pl.ds(base, 16)] = v.astype(jnp.int32)
            return carry
        lax.fori_loop(0, nch, tb, 0)

    temper_all()

    def mt_next(p):
        @pl.when(p >= _MTN)
        def _():
            twist()
            temper_all()
        p = jnp.where(p >= _MTN, 0, p)
        y = _sread(st_v, p).astype(jnp.uint32)
        return y, p + 1

    swap_lanes = lane < 2

    def flat_shuffle(base, count, p):
        iv0 = jnp.broadcast_to(count - 1, (16,))
        mk = iv0
        mk = mk | (mk >> 1)
        mk = mk | (mk >> 2)
        mk = mk | (mk >> 4)
        mk = mk | (mk >> 8)
        mk = mk | (mk >> 16)

        def ocond(c):
            return c[0] > 0

        def obody(c):
            i_s, p_s, i_v, mask_v = c

            @pl.when(p_s >= _MTN)
            def _():
                twist()
                temper_all()

            p_s = jnp.where(p_s >= _MTN, 0, p_s)
            trip = jnp.minimum(i_s, _MTN - p_s)

            def apply_swap(pidx, pmask):
                g = plsc.load_gather(comb_v, [pidx])
                vals = jnp.where(lane == 0, g[1], g[0])
                plsc.store_scatter(comb_v, [pidx], vals, mask=pmask)

            pv0 = jnp.broadcast_to(p_s, (16,))
            y0 = plsc.load_gather(st_v, [pv0])

            def inner(t, ic):
                i_v, mask_v, p_v, y_v, pidx, pmask = ic
                yn = plsc.load_gather(st_v, [p_v + 1])
                apply_swap(pidx, pmask)
                val = y_v & mask_v
                acc = val <= i_v
                j = jnp.minimum(val, i_v)
                nidx = base + jnp.where(lane == 0, i_v, j)
                nmask = jnp.logical_and(acc, swap_lanes)
                i_v = i_v - jnp.where(acc, 1, 0)
                mh = mask_v >> 1
                mask_v = jnp.where(i_v <= mh, mh, mask_v)
                return (i_v, mask_v, p_v + 1, yn, nidx, nmask)

            i_v, mask_v, _, _, pidx, pmask = lax.fori_loop(
                0, trip, inner,
                (i_v, mask_v, pv0, y0,
                 jnp.zeros((16,), jnp.int32), lane < 0))
            apply_swap(pidx, pmask)
            return (i_v[0], p_s + trip, i_v, mask_v)

        i_s, p, _, _ = lax.while_loop(ocond, obody, (count - 1, p, iv0, mk))
        return p

    def comp(c, carry):
        cp, cn = carry
        lab = m_v[pl.ds(c * 16, 16)]
        lidx = c * 16 + lane
        mp = lab == 1
        mn = lab == 0
        plsc.store_compressed(pos_v.at[pl.ds(cp, 16)], lidx, mask=mp)
        plsc.store_compressed(comb_v.at[pl.ds(cn, 16)], lidx, mask=mn)
        cp = cp + plsc.all_reduce_population_count(mp)[0]
        cn = cn + plsc.all_reduce_population_count(mn)[0]
        return (cp, cn)

    n, nn = lax.fori_loop(0, _LP // 16, comp, (jnp.int32(0), jnp.int32(0)))

    def pcopy(c, carry):
        comb_v[pl.ds(nn + c * 16, 16)] = pos_v[pl.ds(c * 16, 16)]
        return carry

    lax.fori_loop(0, (n + 15) // 16, pcopy, 0)

    p0 = _sread(mt_v, _MTN)
    p0 = flat_shuffle(nn, n, p0)
    p0 = flat_shuffle(jnp.int32(0), nn, p0)

    kpos = jnp.minimum(n, _NPOS)
    kneg = jnp.minimum((n.astype(jnp.float32) * _NNEG_T).astype(jnp.int32), nn)

    pidx = comb_v[pl.ds(nn, 16)]
    pm = lane < kpos
    pv = plsc.load_gather(lp_v, [jnp.where(pm, pidx, 0)])
    sp = jnp.sum(jnp.where(pm, pv, 0.0))

    def nsum(c, acc):
        idx = comb_v[pl.ds(c * 16, 16)]
        lm = c * 16 + lane < kneg
        v = plsc.load_gather(ln_v, [jnp.where(lm, idx, 0)])
        return acc + jnp.where(lm, v, 0.0)

    snv = lax.fori_loop(0, (kneg + 15) // 16, nsum, jnp.zeros((16,), jnp.float32))
    sn = jnp.sum(snv)
    cnt = jnp.maximum(kpos + kneg, 1).astype(jnp.float32)
    has = n > 0
    lane = lax.iota(jnp.int32, 16)
    num = jnp.where(lane == 0, sp + sn, jnp.where(lane == 1, lre_num, 0.0))
    den = jnp.where(lane == 0, cnt,
                    jnp.where(lane == 1, jnp.maximum(4 * n, 1).astype(jnp.float32),
                              1.0))
    res = jnp.where(jnp.logical_and(has, lane < 2), num / den, 0.0)
    res_v[...] = res
    pltpu.sync_copy(res_v, out_hbm.at[wid])


def kernel(pred_cls, pred_reg, gt_bbox):
    cls = pred_cls.reshape(_B, _L)
    cls = jnp.pad(cls, ((0, 0), (0, _LP - _L)), constant_values=0.5)
    reg = pred_reg.reshape(4 * _B, _L)
    reg = jnp.pad(reg, ((0, 0), (0, _LP - _L)))
    gt4 = jnp.repeat(gt_bbox, 4, axis=0)
    anc = jnp.asarray(_ANC)
    m, lp, ln, lre = pl.pallas_call(
        _tc_body,
        out_shape=[
            jax.ShapeDtypeStruct((_B, _LP), jnp.int32),
            jax.ShapeDtypeStruct((_B, _LP), jnp.float32),
            jax.ShapeDtypeStruct((_B, _LP), jnp.float32),
            jax.ShapeDtypeStruct((4 * _B, 16), jnp.float32),
        ],
    )(cls, reg, gt_bbox, gt4, anc)

    info = plsc.get_sparse_core_info()
    nc = info.num_cores
    mesh = plsc.VectorSubcoreMesh(core_axis_name="c", subcore_axis_name="s")
    sc = functools.partial(
        pl.kernel,
        mesh=mesh,
        compiler_params=pltpu.CompilerParams(needs_layout_passes=False),
        out_type=jax.ShapeDtypeStruct((_B, 16), jnp.float32),
        scratch_types=[
            pltpu.VMEM((_LP,), jnp.int32),
            pltpu.VMEM((_LP,), jnp.float32),
            pltpu.VMEM((_LP,), jnp.float32),
            pltpu.VMEM((_LP,), jnp.int32),
            pltpu.VMEM((_LP,), jnp.int32),
            pltpu.VMEM((_MTP,), jnp.int32),
            pltpu.VMEM((_MTP,), jnp.int32),
            pltpu.VMEM((4, 16), jnp.float32),
            pltpu.VMEM((16,), jnp.float32),
        ],
    )(functools.partial(_sc_body, nc))
    out = sc(m, lp, ln, lre, jnp.asarray(_MT_ROWS))

    loss_cls = jnp.sum(out[:, 0])
    loss_reg = jnp.sum(out[:, 1])
    total = _CLS_W * loss_cls + _REG_W * loss_reg
    return (total.astype(jnp.float32), loss_cls.astype(jnp.float32),
            loss_reg.astype(jnp.float32))


# speedup vs baseline: 1.2609x; 1.2609x over previous
import functools

import numpy as np
import jax
import jax.numpy as jnp
from jax import lax
from jax.experimental import pallas as pl
from jax.experimental.pallas import tpu as pltpu
from jax.experimental.pallas import tpu_sc as plsc

_CLS_W = 1.0
_REG_W = 1.0
_STRIDE = 8
_SEARCH = 255
_FEAT = 17
_RATIOS = [0.33, 0.5, 1.0, 2.0, 3.0]
_SCALES = [8]
_B = 32
_A = len(_RATIOS) * len(_SCALES)
_L = _A * _FEAT * _FEAT
_LP = 1536
_NPOS = 16
_NNEG_T = 3.0
_MTN = 624
_MTP = 632


def _build_anchor_rows():
    n = _A
    na = np.zeros((n, 4), np.float32)
    size = _STRIDE * _STRIDE
    c = 0
    for r in _RATIOS:
        ws = int(np.sqrt(size / r))
        hs = int(ws * r)
        for s in _SCALES:
            na[c] = [0.0, 0.0, ws * s, hs * s]
            c += 1
    ori = _SEARCH // 2 - (_FEAT // 2) * _STRIDE
    xs = (ori + _STRIDE * np.arange(_FEAT)).astype(np.float32)
    xx, yy = np.meshgrid(xs, xs)
    cx = np.tile(xx[None], (n, 1, 1))
    cy = np.tile(yy[None], (n, 1, 1))
    w = np.tile(na[:, 2, None, None], (1, _FEAT, _FEAT))
    h = np.tile(na[:, 3, None, None], (1, _FEAT, _FEAT))
    corner = np.stack([cx - w / 2, cy - h / 2, cx + w / 2, cy + h / 2], 0).reshape(4, -1)
    center = np.stack([cx, cy, w, h], 0).reshape(4, -1)
    anc = np.zeros((8, _LP), np.float32)
    anc[0:4, :_L] = corner
    anc[4:8, :_L] = center
    anc[6, _L:] = 1.0
    anc[7, _L:] = 1.0
    return anc


def _build_mt_rows():
    rows = np.zeros((_B, _MTP), np.uint32)
    for i in range(_B):
        st = np.random.RandomState(i).get_state()
        rows[i, :_MTN] = st[1].astype(np.uint32)
        rows[i, _MTN] = np.uint32(st[2])
    return rows.view(np.int32)


_ANC = _build_anchor_rows()
_MT_ROWS = _build_mt_rows()


def _sl1(x):
    d = jnp.abs(x)
    return jnp.where(d < 1.0, 0.5 * d * d, d - 0.5)


def _iou_labels(g_ref, anc_ref):
    gx1 = g_ref[:, 0:1]
    gy1 = g_ref[:, 1:2]
    gx2 = g_ref[:, 2:3]
    gy2 = g_ref[:, 3:4]
    ax1 = anc_ref[0:1, :]
    ay1 = anc_ref[1:2, :]
    ax2 = anc_ref[2:3, :]
    ay2 = anc_ref[3:4, :]
    ix = jnp.clip(jnp.minimum(gx2, ax2) - jnp.maximum(gx1, ax1), 0.0, None)
    iy = jnp.clip(jnp.minimum(gy2, ay2) - jnp.maximum(gy1, ay1), 0.0, None)
    inter = ix * iy
    area_g = (gx2 - gx1) * (gy2 - gy1)
    area_a = (ax2 - ax1) * (ay2 - ay1)
    iou = inter / (area_g + area_a - inter)
    return jnp.where(iou > 0.6, 1, jnp.where(iou < 0.3, 0, -1)).astype(jnp.int32)


def _tc_body(cls_ref, r0_ref, r1_ref, r2_ref, r3_ref, g_ref, anc_ref,
             m_ref, lp_ref, ln_ref, lre_ref):
    lab = _iou_labels(g_ref, anc_ref)
    lidx = lax.broadcasted_iota(jnp.int32, (_B, _LP), 1)
    m_ref[...] = jnp.where(lidx < _L, lab, -1)
    c = jnp.clip(cls_ref[...], 1e-7, 1.0 - 1e-7)
    lp_ref[...] = -jnp.log(c)
    ln_ref[...] = -jnp.log(1.0 - c)
    gx1 = g_ref[:, 0:1]
    gy1 = g_ref[:, 1:2]
    gx2 = g_ref[:, 2:3]
    gy2 = g_ref[:, 3:4]
    acx = anc_ref[4:5, :]
    acy = anc_ref[5:6, :]
    acw = anc_ref[6:7, :]
    ach = anc_ref[7:8, :]
    e0 = (gx1 - acx) / acw
    e1 = (gy1 - acy) / ach
    e2 = (gx2 - acx) / acw
    e3 = (gy2 - acy) / ach
    sl = (_sl1(r0_ref[...] - e0) + _sl1(r1_ref[...] - e1)
          + _sl1(r2_ref[...] - e2) + _sl1(r3_ref[...] - e3))
    msl = jnp.where(lab == 1, sl, 0.0)
    lre_ref[...] = jnp.broadcast_to(jnp.sum(msl, axis=1, keepdims=True),
                                    (_B, 16))


def _sread(ref, i):
    idxv = jnp.broadcast_to(jnp.asarray(i, jnp.int32), (16,))
    return plsc.load_gather(ref, [idxv])[0]


def _swrite(ref, i, val):
    idxv = jnp.broadcast_to(jnp.asarray(i, jnp.int32), (16,))
    valv = jnp.broadcast_to(val, (16,))
    mask = lax.iota(jnp.int32, 16) == 0
    plsc.store_scatter(ref, [idxv], valv, mask=mask)


def _sc_body(nc, m_hbm, lp_hbm, ln_hbm, lre_hbm, mt_hbm, out_hbm,
             m_v, lp_v, ln_v, pos_v, comb_v, mt_v, st_v, res_v):
    wid = lax.axis_index("s") * nc + lax.axis_index("c")
    pltpu.sync_copy(m_hbm.at[wid], m_v)
    pltpu.sync_copy(lp_hbm.at[wid], lp_v)
    pltpu.sync_copy(ln_hbm.at[wid], ln_v)
    pltpu.sync_copy(lre_hbm.at[wid], res_v)
    pltpu.sync_copy(mt_hbm.at[wid], mt_v)
    lre_num = res_v[...][0]

    lane = lax.iota(jnp.int32, 16)
    u1 = jnp.uint32(1)
    up = jnp.uint32(0x80000000)
    lo = jnp.uint32(0x7FFFFFFF)
    mag = jnp.uint32(0x9908B0DF)
    z = jnp.uint32(0)
    nch = _MTN // 16

    def temper_all():
        def tp(c, carry):
            y = mt_v[pl.ds(c * 16, 16)].astype(jnp.uint32)
            y = y ^ (y >> jnp.uint32(11))
            y = y ^ ((y << jnp.uint32(7)) & jnp.uint32(0x9D2C5680))
            y = y ^ ((y << jnp.uint32(15)) & jnp.uint32(0xEFC60000))
            y = y ^ (y >> jnp.uint32(18))
            st_v[pl.ds(c * 16, 16)] = y.astype(jnp.int32)
            return carry
        lax.fori_loop(0, nch, tp, 0)

    def twist():
        def tb(c, carry):
            base = c * 16
            iv = base + lane
            cur = mt_v[pl.ds(base, 16)].astype(jnp.uint32)
            nxt = jnp.where(iv == _MTN - 1, 0, iv + 1)
            far = jnp.where(iv < _MTN - 397, iv + 397, iv - (_MTN - 397))
            nv = plsc.load_gather(mt_v, [nxt]).astype(jnp.uint32)
            fv = plsc.load_gather(mt_v, [far]).astype(jnp.uint32)
            y = (cur & up) | (nv & lo)
            v = fv ^ (y >> u1) ^ jnp.where((y & u1) != z, mag, z)
            mt_v[pl.ds(base, 16)] = v.astype(jnp.int32)
            return carry
        lax.fori_loop(0, nch, tb, 0)

    temper_all()

    def mt_next(p):
        @pl.when(p >= _MTN)
        def _():
            twist()
            temper_all()
        p = jnp.where(p >= _MTN, 0, p)
        y = _sread(st_v, p).astype(jnp.uint32)
        return y, p + 1

    swap_lanes = lane < 2

    def flat_shuffle(base, count, p):
        iv0 = jnp.broadcast_to(count - 1, (16,))
        mk = iv0
        mk = mk | (mk >> 1)
        mk = mk | (mk >> 2)
        mk = mk | (mk >> 4)
        mk = mk | (mk >> 8)
        mk = mk | (mk >> 16)

        def ocond(c):
            return c[0] > 0

        def obody(c):
            i_s, p_s, i_v, mask_v = c

            @pl.when(p_s >= _MTN)
            def _():
                twist()
                temper_all()

            p_s = jnp.where(p_s >= _MTN, 0, p_s)
            trip = jnp.minimum(i_s, _MTN - p_s)

            def apply_swap(pidx, pmask):
                g = plsc.load_gather(comb_v, [pidx])
                vals = jnp.where(lane == 0, g[1], g[0])
                plsc.store_scatter(comb_v, [pidx], vals, mask=pmask)

            pv0 = jnp.broadcast_to(p_s, (16,))
            y0 = plsc.load_gather(st_v, [pv0])

            def inner(t, ic):
                i_v, mask_v, p_v, y_v, pidx, pmask = ic
                yn = plsc.load_gather(st_v, [p_v + 1])
                apply_swap(pidx, pmask)
                val = y_v & mask_v
                acc = val <= i_v
                j = jnp.minimum(val, i_v)
                nidx = base + jnp.where(lane == 0, i_v, j)
                nmask = jnp.logical_and(acc, swap_lanes)
                i_v = i_v - jnp.where(acc, 1, 0)
                mh = mask_v >> 1
                mask_v = jnp.where(i_v <= mh, mh, mask_v)
                return (i_v, mask_v, p_v + 1, yn, nidx, nmask)

            i_v, mask_v, _, _, pidx, pmask = lax.fori_loop(
                0, trip, inner,
                (i_v, mask_v, pv0, y0,
                 jnp.zeros((16,), jnp.int32), lane < 0))
            apply_swap(pidx, pmask)
            return (i_v[0], p_s + trip, i_v, mask_v)

        i_s, p, _, _ = lax.while_loop(ocond, obody, (count - 1, p, iv0, mk))
        return p

    def comp(c, carry):
        cp, cn = carry
        lab = m_v[pl.ds(c * 16, 16)]
        lidx = c * 16 + lane
        mp = lab == 1
        mn = lab == 0
        plsc.store_compressed(pos_v.at[pl.ds(cp, 16)], lidx, mask=mp)
        plsc.store_compressed(comb_v.at[pl.ds(cn, 16)], lidx, mask=mn)
        cp = cp + plsc.all_reduce_population_count(mp)[0]
        cn = cn + plsc.all_reduce_population_count(mn)[0]
        return (cp, cn)

    n, nn = lax.fori_loop(0, _LP // 16, comp, (jnp.int32(0), jnp.int32(0)))

    def pcopy(c, carry):
        comb_v[pl.ds(nn + c * 16, 16)] = pos_v[pl.ds(c * 16, 16)]
        return carry

    lax.fori_loop(0, (n + 15) // 16, pcopy, 0)

    p0 = _sread(mt_v, _MTN)
    p0 = flat_shuffle(nn, n, p0)
    p0 = flat_shuffle(jnp.int32(0), nn, p0)

    kpos = jnp.minimum(n, _NPOS)
    kneg = jnp.minimum((n.astype(jnp.float32) * _NNEG_T).astype(jnp.int32), nn)

    pidx = comb_v[pl.ds(nn, 16)]
    pm = lane < kpos
    pv = plsc.load_gather(lp_v, [jnp.where(pm, pidx, 0)])
    sp = jnp.sum(jnp.where(pm, pv, 0.0))

    def nsum(c, acc):
        idx = comb_v[pl.ds(c * 16, 16)]
        lm = c * 16 + lane < kneg
        v = plsc.load_gather(ln_v, [jnp.where(lm, idx, 0)])
        return acc + jnp.where(lm, v, 0.0)

    snv = lax.fori_loop(0, (kneg + 15) // 16, nsum, jnp.zeros((16,), jnp.float32))
    sn = jnp.sum(snv)
    cnt = jnp.maximum(kpos + kneg, 1).astype(jnp.float32)
    has = n > 0
    lane = lax.iota(jnp.int32, 16)
    num = jnp.where(lane == 0, sp + sn, jnp.where(lane == 1, lre_num, 0.0))
    den = jnp.where(lane == 0, cnt,
                    jnp.where(lane == 1, jnp.maximum(4 * n, 1).astype(jnp.float32),
                              1.0))
    res = jnp.where(jnp.logical_and(has, lane < 2), num / den, 0.0)
    res_v[...] = res
    pltpu.sync_copy(res_v, out_hbm.at[wid])


def kernel(pred_cls, pred_reg, gt_bbox):
    cls = pred_cls.reshape(_B, _L)
    cls = jnp.pad(cls, ((0, 0), (0, _LP - _L)), constant_values=0.5)
    reg = pred_reg.reshape(_B, 4, _L)
    reg = jnp.pad(reg, ((0, 0), (0, 0), (0, _LP - _L)))
    anc = jnp.asarray(_ANC)
    m, lp, ln, lre = pl.pallas_call(
        _tc_body,
        out_shape=[
            jax.ShapeDtypeStruct((_B, _LP), jnp.int32),
            jax.ShapeDtypeStruct((_B, _LP), jnp.float32),
            jax.ShapeDtypeStruct((_B, _LP), jnp.float32),
            jax.ShapeDtypeStruct((_B, 16), jnp.float32),
        ],
    )(cls, reg[:, 0], reg[:, 1], reg[:, 2], reg[:, 3], gt_bbox, anc)

    info = plsc.get_sparse_core_info()
    nc = info.num_cores
    mesh = plsc.VectorSubcoreMesh(core_axis_name="c", subcore_axis_name="s")
    sc = functools.partial(
        pl.kernel,
        mesh=mesh,
        compiler_params=pltpu.CompilerParams(needs_layout_passes=False),
        out_type=jax.ShapeDtypeStruct((_B, 16), jnp.float32),
        scratch_types=[
            pltpu.VMEM((_LP,), jnp.int32),
            pltpu.VMEM((_LP,), jnp.float32),
            pltpu.VMEM((_LP,), jnp.float32),
            pltpu.VMEM((_LP,), jnp.int32),
            pltpu.VMEM((_LP,), jnp.int32),
            pltpu.VMEM((_MTP,), jnp.int32),
            pltpu.VMEM((_MTP,), jnp.int32),
            pltpu.VMEM((16,), jnp.float32),
        ],
    )(functools.partial(_sc_body, nc))
    out = sc(m, lp, ln, lre, jnp.asarray(_MT_ROWS))

    loss_cls = jnp.sum(out[:, 0])
    loss_reg = jnp.sum(out[:, 1])
    total = _CLS_W * loss_cls + _REG_W * loss_reg
    return (total.astype(jnp.float32), loss_cls.astype(jnp.float32),
            loss_reg.astype(jnp.float32))


# speedup vs baseline: 1.3059x; 1.0357x over previous
import functools

import numpy as np
import jax
import jax.numpy as jnp
from jax import lax
from jax.experimental import pallas as pl
from jax.experimental.pallas import tpu as pltpu
from jax.experimental.pallas import tpu_sc as plsc

_CLS_W = 1.0
_REG_W = 1.0
_STRIDE = 8
_SEARCH = 255
_FEAT = 17
_RATIOS = [0.33, 0.5, 1.0, 2.0, 3.0]
_SCALES = [8]
_B = 32
_A = len(_RATIOS) * len(_SCALES)
_L = _A * _FEAT * _FEAT
_LP = 1536
_NPOS = 16
_NNEG_T = 3.0
_MTN = 624
_MTP = 632


def _build_anchor_rows():
    n = _A
    na = np.zeros((n, 4), np.float32)
    size = _STRIDE * _STRIDE
    c = 0
    for r in _RATIOS:
        ws = int(np.sqrt(size / r))
        hs = int(ws * r)
        for s in _SCALES:
            na[c] = [0.0, 0.0, ws * s, hs * s]
            c += 1
    ori = _SEARCH // 2 - (_FEAT // 2) * _STRIDE
    xs = (ori + _STRIDE * np.arange(_FEAT)).astype(np.float32)
    xx, yy = np.meshgrid(xs, xs)
    cx = np.tile(xx[None], (n, 1, 1))
    cy = np.tile(yy[None], (n, 1, 1))
    w = np.tile(na[:, 2, None, None], (1, _FEAT, _FEAT))
    h = np.tile(na[:, 3, None, None], (1, _FEAT, _FEAT))
    corner = np.stack([cx - w / 2, cy - h / 2, cx + w / 2, cy + h / 2], 0).reshape(4, -1)
    center = np.stack([cx, cy, w, h], 0).reshape(4, -1)
    anc = np.zeros((8, _LP), np.float32)
    anc[0:4, :_L] = corner
    anc[4:8, :_L] = center
    anc[6, _L:] = 1.0
    anc[7, _L:] = 1.0
    return anc


def _build_mt_rows():
    rows = np.zeros((_B, _MTP), np.uint32)
    for i in range(_B):
        st = np.random.RandomState(i).get_state()
        rows[i, :_MTN] = st[1].astype(np.uint32)
        rows[i, _MTN] = np.uint32(st[2])
    return rows.view(np.int32)


_ANC = _build_anchor_rows()
_MT_ROWS = _build_mt_rows()


def _sl1(x):
    d = jnp.abs(x)
    return jnp.where(d < 1.0, 0.5 * d * d, d - 0.5)


def _iou_labels(g_ref, anc_ref):
    gx1 = g_ref[:, 0:1]
    gy1 = g_ref[:, 1:2]
    gx2 = g_ref[:, 2:3]
    gy2 = g_ref[:, 3:4]
    ax1 = anc_ref[0:1, :]
    ay1 = anc_ref[1:2, :]
    ax2 = anc_ref[2:3, :]
    ay2 = anc_ref[3:4, :]
    ix = jnp.clip(jnp.minimum(gx2, ax2) - jnp.maximum(gx1, ax1), 0.0, None)
    iy = jnp.clip(jnp.minimum(gy2, ay2) - jnp.maximum(gy1, ay1), 0.0, None)
    inter = ix * iy
    area_g = (gx2 - gx1) * (gy2 - gy1)
    area_a = (ax2 - ax1) * (ay2 - ay1)
    iou = inter / (area_g + area_a - inter)
    return jnp.where(iou > 0.6, 1, jnp.where(iou < 0.3, 0, -1)).astype(jnp.int32)


def _tc_body(cls_ref, r0_ref, r1_ref, r2_ref, r3_ref, g_ref, anc_ref,
             m_ref, lp_ref, ln_ref, lre_ref):
    lab = _iou_labels(g_ref, anc_ref)
    lidx = lax.broadcasted_iota(jnp.int32, (_B, _LP), 1)
    m_ref[...] = jnp.where(lidx < _L, lab, -1)
    c = jnp.clip(cls_ref[...], 1e-7, 1.0 - 1e-7)
    lp_ref[...] = -jnp.log(c)
    ln_ref[...] = -jnp.log(1.0 - c)
    gx1 = g_ref[:, 0:1]
    gy1 = g_ref[:, 1:2]
    gx2 = g_ref[:, 2:3]
    gy2 = g_ref[:, 3:4]
    acx = anc_ref[4:5, :]
    acy = anc_ref[5:6, :]
    acw = anc_ref[6:7, :]
    ach = anc_ref[7:8, :]
    e0 = (gx1 - acx) / acw
    e1 = (gy1 - acy) / ach
    e2 = (gx2 - acx) / acw
    e3 = (gy2 - acy) / ach
    sl = (_sl1(r0_ref[...] - e0) + _sl1(r1_ref[...] - e1)
          + _sl1(r2_ref[...] - e2) + _sl1(r3_ref[...] - e3))
    msl = jnp.where(lab == 1, sl, 0.0)
    lre_ref[...] = jnp.broadcast_to(jnp.sum(msl, axis=1, keepdims=True),
                                    (_B, 16))


def _sread(ref, i):
    idxv = jnp.broadcast_to(jnp.asarray(i, jnp.int32), (16,))
    return plsc.load_gather(ref, [idxv])[0]


def _swrite(ref, i, val):
    idxv = jnp.broadcast_to(jnp.asarray(i, jnp.int32), (16,))
    valv = jnp.broadcast_to(val, (16,))
    mask = lax.iota(jnp.int32, 16) == 0
    plsc.store_scatter(ref, [idxv], valv, mask=mask)


def _sc_body(nc, m_hbm, lp_hbm, ln_hbm, lre_hbm, mt_hbm, out_hbm,
             m_v, lp_v, ln_v, pos_v, comb_v, mt_v, st_v, res_v,
             sem_a, sem_b):
    wid = lax.axis_index("s") * nc + lax.axis_index("c")
    cpy_m = pltpu.async_copy(m_hbm.at[wid], m_v, sem_a)
    cpy_lp = pltpu.async_copy(lp_hbm.at[wid], lp_v, sem_b)
    cpy_ln = pltpu.async_copy(ln_hbm.at[wid], ln_v, sem_b)
    cpy_lre = pltpu.async_copy(lre_hbm.at[wid], res_v, sem_b)
    pltpu.sync_copy(mt_hbm.at[wid], mt_v)

    lane = lax.iota(jnp.int32, 16)
    u1 = jnp.uint32(1)
    up = jnp.uint32(0x80000000)
    lo = jnp.uint32(0x7FFFFFFF)
    mag = jnp.uint32(0x9908B0DF)
    z = jnp.uint32(0)
    nch = _MTN // 16

    def temper_all():
        def tp(c, carry):
            y = mt_v[pl.ds(c * 16, 16)].astype(jnp.uint32)
            y = y ^ (y >> jnp.uint32(11))
            y = y ^ ((y << jnp.uint32(7)) & jnp.uint32(0x9D2C5680))
            y = y ^ ((y << jnp.uint32(15)) & jnp.uint32(0xEFC60000))
            y = y ^ (y >> jnp.uint32(18))
            st_v[pl.ds(c * 16, 16)] = y.astype(jnp.int32)
            return carry
        lax.fori_loop(0, nch, tp, 0)

    def twist():
        def tb(c, carry):
            base = c * 16
            iv = base + lane
            cur = mt_v[pl.ds(base, 16)].astype(jnp.uint32)
            nxt = jnp.where(iv == _MTN - 1, 0, iv + 1)
            far = jnp.where(iv < _MTN - 397, iv + 397, iv - (_MTN - 397))
            nv = plsc.load_gather(mt_v, [nxt]).astype(jnp.uint32)
            fv = plsc.load_gather(mt_v, [far]).astype(jnp.uint32)
            y = (cur & up) | (nv & lo)
            v = fv ^ (y >> u1) ^ jnp.where((y & u1) != z, mag, z)
            mt_v[pl.ds(base, 16)] = v.astype(jnp.int32)
            t = v ^ (v >> jnp.uint32(11))
            t = t ^ ((t << jnp.uint32(7)) & jnp.uint32(0x9D2C5680))
            t = t ^ ((t << jnp.uint32(15)) & jnp.uint32(0xEFC60000))
            t = t ^ (t >> jnp.uint32(18))
            st_v[pl.ds(base, 16)] = t.astype(jnp.int32)
            return carry
        lax.fori_loop(0, nch, tb, 0)

    swap_lanes = lane < 2

    def flat_shuffle(base, count, p):
        iv0 = jnp.broadcast_to(count - 1, (16,))
        mk = iv0
        mk = mk | (mk >> 1)
        mk = mk | (mk >> 2)
        mk = mk | (mk >> 4)
        mk = mk | (mk >> 8)
        mk = mk | (mk >> 16)

        def ocond(c):
            return c[0] > 0

        def obody(c):
            i_s, p_s, i_v, mask_v = c

            @pl.when(p_s >= _MTN)
            def _():
                twist()

            p_s = jnp.where(p_s >= _MTN, 0, p_s)
            trip = jnp.minimum(i_s, _MTN - p_s)

            def apply_swap(pidx, pmask):
                g = plsc.load_gather(comb_v, [pidx])
                vals = jnp.where(lane == 0, g[1], g[0])
                plsc.store_scatter(comb_v, [pidx], vals, mask=pmask)

            pv0 = jnp.broadcast_to(p_s, (16,))
            y0 = plsc.load_gather(st_v, [pv0])

            def inner(t, ic):
                i_v, mask_v, p_v, y_v, pidx, pmask = ic
                yn = plsc.load_gather(st_v, [p_v + 1])
                apply_swap(pidx, pmask)
                val = y_v & mask_v
                acc = val <= i_v
                j = jnp.minimum(val, i_v)
                nidx = base + jnp.where(lane == 0, i_v, j)
                nmask = jnp.logical_and(acc, swap_lanes)
                i_v = i_v - jnp.where(acc, 1, 0)
                mh = mask_v >> 1
                mask_v = jnp.where(i_v <= mh, mh, mask_v)
                return (i_v, mask_v, p_v + 1, yn, nidx, nmask)

            i_v, mask_v, _, _, pidx, pmask = lax.fori_loop(
                0, trip, inner,
                (i_v, mask_v, pv0, y0,
                 jnp.zeros((16,), jnp.int32), lane < 0))
            apply_swap(pidx, pmask)
            return (i_v[0], p_s + trip, i_v, mask_v)

        i_s, p, _, _ = lax.while_loop(ocond, obody, (count - 1, p, iv0, mk))
        return p

    def comp(c, carry):
        cp, cn = carry
        lab = m_v[pl.ds(c * 16, 16)]
        lidx = c * 16 + lane
        mp = lab == 1
        mn = lab == 0
        plsc.store_compressed(pos_v.at[pl.ds(cp, 16)], lidx, mask=mp)
        plsc.store_compressed(comb_v.at[pl.ds(cn, 16)], lidx, mask=mn)
        cp = cp + plsc.all_reduce_population_count(mp)[0]
        cn = cn + plsc.all_reduce_population_count(mn)[0]
        return (cp, cn)

    p0 = _sread(mt_v, _MTN)

    @pl.when(p0 < _MTN)
    def _():
        temper_all()

    cpy_m.wait()
    n, nn = lax.fori_loop(0, _LP // 16, comp, (jnp.int32(0), jnp.int32(0)))

    def pcopy(c, carry):
        comb_v[pl.ds(nn + c * 16, 16)] = pos_v[pl.ds(c * 16, 16)]
        return carry

    lax.fori_loop(0, (n + 15) // 16, pcopy, 0)

    p0 = flat_shuffle(nn, n, p0)
    p0 = flat_shuffle(jnp.int32(0), nn, p0)

    cpy_lp.wait()
    cpy_ln.wait()
    cpy_lre.wait()
    lre_num = res_v[...][0]
    kpos = jnp.minimum(n, _NPOS)
    kneg = jnp.minimum((n.astype(jnp.float32) * _NNEG_T).astype(jnp.int32), nn)

    pidx = comb_v[pl.ds(nn, 16)]
    pm = lane < kpos
    pv = plsc.load_gather(lp_v, [jnp.where(pm, pidx, 0)])
    sp = jnp.sum(jnp.where(pm, pv, 0.0))

    def nsum(c, acc):
        idx = comb_v[pl.ds(c * 16, 16)]
        lm = c * 16 + lane < kneg
        v = plsc.load_gather(ln_v, [jnp.where(lm, idx, 0)])
        return acc + jnp.where(lm, v, 0.0)

    snv = lax.fori_loop(0, (kneg + 15) // 16, nsum, jnp.zeros((16,), jnp.float32))
    sn = jnp.sum(snv)
    cnt = jnp.maximum(kpos + kneg, 1).astype(jnp.float32)
    has = n > 0
    lane = lax.iota(jnp.int32, 16)
    num = jnp.where(lane == 0, sp + sn, jnp.where(lane == 1, lre_num, 0.0))
    den = jnp.where(lane == 0, cnt,
                    jnp.where(lane == 1, jnp.maximum(4 * n, 1).astype(jnp.float32),
                              1.0))
    res = jnp.where(jnp.logical_and(has, lane < 2), num / den, 0.0)
    res_v[...] = res
    pltpu.sync_copy(res_v, out_hbm.at[wid])


def kernel(pred_cls, pred_reg, gt_bbox):
    cls = pred_cls.reshape(_B, _L)
    cls = jnp.pad(cls, ((0, 0), (0, _LP - _L)), constant_values=0.5)
    reg = pred_reg.reshape(_B, 4, _L)
    reg = jnp.pad(reg, ((0, 0), (0, 0), (0, _LP - _L)))
    anc = jnp.asarray(_ANC)
    m, lp, ln, lre = pl.pallas_call(
        _tc_body,
        out_shape=[
            jax.ShapeDtypeStruct((_B, _LP), jnp.int32),
            jax.ShapeDtypeStruct((_B, _LP), jnp.float32),
            jax.ShapeDtypeStruct((_B, _LP), jnp.float32),
            jax.ShapeDtypeStruct((_B, 16), jnp.float32),
        ],
    )(cls, reg[:, 0], reg[:, 1], reg[:, 2], reg[:, 3], gt_bbox, anc)

    info = plsc.get_sparse_core_info()
    nc = info.num_cores
    mesh = plsc.VectorSubcoreMesh(core_axis_name="c", subcore_axis_name="s")
    sc = functools.partial(
        pl.kernel,
        mesh=mesh,
        compiler_params=pltpu.CompilerParams(needs_layout_passes=False),
        out_type=jax.ShapeDtypeStruct((_B, 16), jnp.float32),
        scratch_types=[
            pltpu.VMEM((_LP,), jnp.int32),
            pltpu.VMEM((_LP,), jnp.float32),
            pltpu.VMEM((_LP,), jnp.float32),
            pltpu.VMEM((_LP,), jnp.int32),
            pltpu.VMEM((_LP,), jnp.int32),
            pltpu.VMEM((_MTP,), jnp.int32),
            pltpu.VMEM((_MTP,), jnp.int32),
            pltpu.VMEM((16,), jnp.float32),
            pltpu.SemaphoreType.DMA,
            pltpu.SemaphoreType.DMA,
        ],
    )(functools.partial(_sc_body, nc))
    out = sc(m, lp, ln, lre, jnp.asarray(_MT_ROWS))

    loss_cls = jnp.sum(out[:, 0])
    loss_reg = jnp.sum(out[:, 1])
    total = _CLS_W * loss_cls + _REG_W * loss_reg
    return (total.astype(jnp.float32), loss_cls.astype(jnp.float32),
            loss_reg.astype(jnp.float32))
